# baseline (speedup 1.0000x reference)
import jax
import jax.numpy as jnp
from jax.experimental import pallas as pl
from jax.experimental.pallas import tpu as pltpu

H = 32
W_ = 32
P = H * W_
CIN = 64
COUT = 64
KK = 9


def _body(x_ref, w_ref, idx_ref, out_ref, s_ref, z_ref):
    @pl.when(pl.program_id(0) == 0)
    def _build_onehot():
        iq = jax.lax.broadcasted_iota(jnp.int32, (P, P), 0)
        for k in range(KK):
            pk = idx_ref[k]
            s_ref[pl.ds(k * P, P), :] = jnp.where(
                iq == pk, 1.0, 0.0).astype(jnp.bfloat16)

    x = x_ref[0].astype(jnp.bfloat16)
    for k in range(KK):
        z_ref[:, pl.ds(k * P, P)] = jnp.dot(
            w_ref[k], x, preferred_element_type=jnp.float32
        ).astype(jnp.bfloat16)
    out_ref[0] = jnp.dot(
        z_ref[...], s_ref[...], preferred_element_type=jnp.float32)


def kernel(x, W, sample_idx):
    B = x.shape[0]
    wstack = jnp.transpose(W.reshape(COUT, CIN, KK), (2, 0, 1)).astype(
        jnp.bfloat16)
    idx = jnp.transpose(sample_idx.reshape(P, KK), (1, 0)).reshape(KK, 1, P)

    out = pl.pallas_call(
        _body,
        grid=(B,),
        in_specs=[
            pl.BlockSpec((1, CIN, P), lambda b: (b, 0, 0)),
            pl.BlockSpec((KK, COUT, CIN), lambda b: (0, 0, 0)),
            pl.BlockSpec((KK, 1, P), lambda b: (0, 0, 0)),
        ],
        out_specs=pl.BlockSpec((1, COUT, P), lambda b: (b, 0, 0)),
        out_shape=jax.ShapeDtypeStruct((B, COUT, P), jnp.float32),
        scratch_shapes=[
            pltpu.VMEM((KK * P, P), jnp.bfloat16),
            pltpu.VMEM((COUT, KK * P), jnp.bfloat16),
        ],
    )(x.reshape(B, CIN, P), wstack, idx)
    return out.reshape(B, COUT, H, W_)


# speedup vs baseline: 1.8848x; 1.8848x over previous
import jax
import jax.numpy as jnp
from jax.experimental import pallas as pl
from jax.experimental.pallas import tpu as pltpu

H = 32
W_ = 32
P = H * W_
CIN = 64
COUT = 64
KK = 9
BB = 4


def _body(x_ref, w_ref, idx_ref, out_ref, s_ref, z_ref):
    @pl.when(pl.program_id(0) == 0)
    def _build_onehot():
        iq = jax.lax.broadcasted_iota(jnp.int32, (P, P), 0)
        for k in range(KK):
            pk = idx_ref[k]
            s_ref[pl.ds(k * P, P), :] = jnp.where(
                iq == pk, 1.0, 0.0).astype(jnp.bfloat16)

    for b in range(BB):
        x = x_ref[b].astype(jnp.bfloat16)
        for k in range(KK):
            z_ref[pl.ds(b * COUT, COUT), pl.ds(k * P, P)] = jnp.dot(
                w_ref[k], x, preferred_element_type=jnp.float32
            ).astype(jnp.bfloat16)
    out_ref[...] = jnp.dot(
        z_ref[...], s_ref[...], preferred_element_type=jnp.float32
    ).reshape(BB, COUT, P)


def kernel(x, W, sample_idx):
    B = x.shape[0]
    wstack = jnp.transpose(W.reshape(COUT, CIN, KK), (2, 0, 1)).astype(
        jnp.bfloat16)
    idx = jnp.transpose(sample_idx.reshape(P, KK), (1, 0)).reshape(KK, 1, P)

    out = pl.pallas_call(
        _body,
        grid=(B // BB,),
        in_specs=[
            pl.BlockSpec((BB, CIN, P), lambda b: (b, 0, 0)),
            pl.BlockSpec((KK, COUT, CIN), lambda b: (0, 0, 0)),
            pl.BlockSpec((KK, 1, P), lambda b: (0, 0, 0)),
        ],
        out_specs=pl.BlockSpec((BB, COUT, P), lambda b: (b, 0, 0)),
        out_shape=jax.ShapeDtypeStruct((B, COUT, P), jnp.float32),
        scratch_shapes=[
            pltpu.VMEM((KK * P, P), jnp.bfloat16),
            pltpu.VMEM((BB * COUT, KK * P), jnp.bfloat16),
        ],
    )(x.reshape(B, CIN, P), wstack, idx)
    return out.reshape(B, COUT, H, W_)


# speedup vs baseline: 1.9065x; 1.0115x over previous
import jax
import jax.numpy as jnp
from jax.experimental import pallas as pl
from jax.experimental.pallas import tpu as pltpu

H = 32
W_ = 32
P = H * W_
CIN = 64
COUT = 64
KK = 9
BB = 8


def _body(x_ref, w_ref, idx_ref, out_ref, s_ref, z_ref):
    @pl.when(pl.program_id(0) == 0)
    def _build_onehot():
        iq = jax.lax.broadcasted_iota(jnp.int32, (P, P), 0)
        for k in range(KK):
            pk = idx_ref[k]
            s_ref[pl.ds(k * P, P), :] = jnp.where(
                iq == pk, 1.0, 0.0).astype(jnp.bfloat16)

    for b in range(BB):
        x = x_ref[b].astype(jnp.bfloat16)
        for k in range(KK):
            z_ref[pl.ds(b * COUT, COUT), pl.ds(k * P, P)] = jnp.dot(
                w_ref[k], x, preferred_element_type=jnp.float32
            ).astype(jnp.bfloat16)
    out_ref[...] = jnp.dot(
        z_ref[...], s_ref[...], preferred_element_type=jnp.float32
    ).reshape(BB, COUT, P)


def kernel(x, W, sample_idx):
    B = x.shape[0]
    wstack = jnp.transpose(W.reshape(COUT, CIN, KK), (2, 0, 1)).astype(
        jnp.bfloat16)
    idx = jnp.transpose(sample_idx.reshape(P, KK), (1, 0)).reshape(KK, 1, P)

    out = pl.pallas_call(
        _body,
        grid=(B // BB,),
        in_specs=[
            pl.BlockSpec((BB, CIN, P), lambda b: (b, 0, 0)),
            pl.BlockSpec((KK, COUT, CIN), lambda b: (0, 0, 0)),
            pl.BlockSpec((KK, 1, P), lambda b: (0, 0, 0)),
        ],
        out_specs=pl.BlockSpec((BB, COUT, P), lambda b: (b, 0, 0)),
        out_shape=jax.ShapeDtypeStruct((B, COUT, P), jnp.float32),
        scratch_shapes=[
            pltpu.VMEM((KK * P, P), jnp.bfloat16),
            pltpu.VMEM((BB * COUT, KK * P), jnp.bfloat16),
        ],
    )(x.reshape(B, CIN, P), wstack, idx)
    return out.reshape(B, COUT, H, W_)


# speedup vs baseline: 2.0958x; 1.0993x over previous
import jax
import jax.numpy as jnp
from jax.experimental import pallas as pl
from jax.experimental.pallas import tpu as pltpu

H = 32
W_ = 32
P = H * W_
CIN = 64
COUT = 64
KK = 9
BB = 8
PB = 256
QW = 768
NJ = P // PB
QBASE = (0, 0, 256, 256)


def _body(x_ref, w_ref, idxc_ref, out_ref, s_ref, xb_ref, x1_ref, xc_ref):
    @pl.when(pl.program_id(0) == 0)
    def _build_onehot():
        iq = jax.lax.broadcasted_iota(jnp.int32, (QW, KK * PB), 0)
        for j in range(NJ):
            cols = idxc_ref[j]
            s_ref[j] = jnp.where(
                iq + QBASE[j] == cols, 1.0, 0.0).astype(jnp.bfloat16)

    xb_ref[...] = x_ref[...].reshape(BB * CIN, P).astype(jnp.bfloat16)
    for j in range(NJ):
        x1_ref[...] = jnp.dot(
            xb_ref[:, pl.ds(QBASE[j], QW)], s_ref[j],
            preferred_element_type=jnp.float32,
        ).astype(jnp.bfloat16)
        for b in range(BB):
            for k in range(KK):
                xc_ref[pl.ds(k * CIN, CIN), pl.ds(b * PB, PB)] = (
                    x1_ref[pl.ds(b * CIN, CIN), pl.ds(k * PB, PB)])
        oj = jnp.dot(w_ref[...], xc_ref[...],
                     preferred_element_type=jnp.float32)
        for b in range(BB):
            out_ref[b, :, pl.ds(j * PB, PB)] = oj[:, b * PB:(b + 1) * PB]


def kernel(x, W, sample_idx):
    B = x.shape[0]
    w2 = jnp.transpose(W.reshape(COUT, CIN, KK), (0, 2, 1)).reshape(
        COUT, KK * CIN).astype(jnp.bfloat16)
    idxc = jnp.transpose(
        sample_idx.reshape(NJ, PB, KK), (0, 2, 1)).reshape(NJ, 1, KK * PB)

    out = pl.pallas_call(
        _body,
        grid=(B // BB,),
        in_specs=[
            pl.BlockSpec((BB, CIN, P), lambda b: (b, 0, 0)),
            pl.BlockSpec((COUT, KK * CIN), lambda b: (0, 0)),
            pl.BlockSpec((NJ, 1, KK * PB), lambda b: (0, 0, 0)),
        ],
        out_specs=pl.BlockSpec((BB, COUT, P), lambda b: (b, 0, 0)),
        out_shape=jax.ShapeDtypeStruct((B, COUT, P), jnp.float32),
        scratch_shapes=[
            pltpu.VMEM((NJ, QW, KK * PB), jnp.bfloat16),
            pltpu.VMEM((BB * CIN, P), jnp.bfloat16),
            pltpu.VMEM((BB * CIN, KK * PB), jnp.bfloat16),
            pltpu.VMEM((KK * CIN, BB * PB), jnp.bfloat16),
        ],
    )(x.reshape(B, CIN, P), w2, idxc)
    return out.reshape(B, COUT, H, W_)


# speedup vs baseline: 2.1987x; 1.0491x over previous
import jax
import jax.numpy as jnp
from jax.experimental import pallas as pl
from jax.experimental.pallas import tpu as pltpu

H = 32
W_ = 32
P = H * W_
CIN = 64
COUT = 64
KK = 9
BB = 16
PB = 256
QW = 768
NJ = P // PB
QBASE = (0, 0, 256, 256)


def _body(x_ref, w_ref, idxc_ref, out_ref, s_ref, xb_ref, x1_ref, xc_ref):
    @pl.when(pl.program_id(0) == 0)
    def _build_onehot():
        iq = jax.lax.broadcasted_iota(jnp.int32, (QW, KK * PB), 0)
        for j in range(NJ):
            cols = idxc_ref[j]
            s_ref[j] = jnp.where(
                iq + QBASE[j] == cols, 1.0, 0.0).astype(jnp.bfloat16)

    xb_ref[...] = x_ref[...].reshape(BB * CIN, P).astype(jnp.bfloat16)
    for j in range(NJ):
        x1_ref[...] = jnp.dot(
            xb_ref[:, pl.ds(QBASE[j], QW)], s_ref[j],
            preferred_element_type=jnp.float32,
        ).astype(jnp.bfloat16)
        for b in range(BB):
            for k in range(KK):
                xc_ref[pl.ds(k * CIN, CIN), pl.ds(b * PB, PB)] = (
                    x1_ref[pl.ds(b * CIN, CIN), pl.ds(k * PB, PB)])
        oj = jnp.dot(w_ref[...], xc_ref[...],
                     preferred_element_type=jnp.float32)
        for b in range(BB):
            out_ref[b, :, pl.ds(j * PB, PB)] = oj[:, b * PB:(b + 1) * PB]


def kernel(x, W, sample_idx):
    B = x.shape[0]
    w2 = jnp.transpose(W.reshape(COUT, CIN, KK), (0, 2, 1)).reshape(
        COUT, KK * CIN).astype(jnp.bfloat16)
    idxc = jnp.transpose(
        sample_idx.reshape(NJ, PB, KK), (0, 2, 1)).reshape(NJ, 1, KK * PB)

    out = pl.pallas_call(
        _body,
        grid=(B // BB,),
        in_specs=[
            pl.BlockSpec((BB, CIN, P), lambda b: (b, 0, 0)),
            pl.BlockSpec((COUT, KK * CIN), lambda b: (0, 0)),
            pl.BlockSpec((NJ, 1, KK * PB), lambda b: (0, 0, 0)),
        ],
        out_specs=pl.BlockSpec((BB, COUT, P), lambda b: (b, 0, 0)),
        out_shape=jax.ShapeDtypeStruct((B, COUT, P), jnp.float32),
        scratch_shapes=[
            pltpu.VMEM((NJ, QW, KK * PB), jnp.bfloat16),
            pltpu.VMEM((BB * CIN, P), jnp.bfloat16),
            pltpu.VMEM((BB * CIN, KK * PB), jnp.bfloat16),
            pltpu.VMEM((KK * CIN, BB * PB), jnp.bfloat16),
        ],
    )(x.reshape(B, CIN, P), w2, idxc)
    return out.reshape(B, COUT, H, W_)
